# baseline (speedup 1.0000x reference)
import dataclasses

import jax
import jax.numpy as jnp
from jax import lax
from jax.experimental import pallas as pl
from jax.experimental.pallas import tpu as pltpu
from jax.experimental.pallas import tpu_sc as plsc

B = 16384
L = 50
D = 32
H = D // 2

NUM_CORES = 2
NUM_SUBCORES = 16
NW = NUM_CORES * NUM_SUBCORES
CPW = B // NW
CHUNK = 64
NCHUNK = CPW // CHUNK
L_PAD = 56


def _tree_sum(xs):
    while len(xs) > 1:
        ys = [xs[i] + xs[i + 1] for i in range(0, len(xs) - 1, 2)]
        if len(xs) % 2:
            ys.append(xs[-1])
        xs = ys
    return xs[0]


def _embed_mean_body(words_hbm, table_hbm, out_hbm, idx_v, rows_v, out_v,
                     outt_v, t0_v, sem):
    wid = lax.axis_index("s") * NUM_CORES + lax.axis_index("c")
    pltpu.sync_copy(table_hbm.at[pl.ds(0, 1)], t0_v)
    t0_lo = t0_v[0, pl.ds(0, H)]
    t0_hi = t0_v[0, pl.ds(H, H)]
    lanes = lax.iota(jnp.int32, 16)
    scale = jnp.float32(1.0 / L)

    @pl.loop(0, NCHUNK)
    def _chunk(c):
        bc = wid * CPW + c * CHUNK
        pltpu.sync_copy(words_hbm.at[:, pl.ds(bc, CHUNK)],
                        idx_v.at[pl.ds(0, L), :])

        @pl.loop(0, L)
        def _fire(l):
            pltpu.async_copy(table_hbm.at[idx_v.at[l]],
                             rows_v.at[pl.ds(l * CHUNK, CHUNK)], sem)

        pltpu.make_async_copy(table_hbm.at[pl.ds(0, L * CHUNK)], rows_v,
                              sem).wait()

        @pl.loop(0, CHUNK)
        def _col(k):
            lo = [rows_v[j * CHUNK + k, pl.ds(0, H)] for j in range(L)]
            hi = [rows_v[j * CHUNK + k, pl.ds(H, H)] for j in range(L)]
            acc_lo = _tree_sum(lo)
            acc_hi = _tree_sum(hi)
            nz = jnp.float32(0.0)
            for q in range(4):
                lrow = q * 16 + lanes
                kcol = jnp.full((16,), 0, jnp.int32) + k
                if (q + 1) * 16 <= L:
                    vals = plsc.load_gather(idx_v, [lrow, kcol])
                    hit = vals == 0
                else:
                    live = lanes < jnp.int32(L - q * 16)
                    vals = plsc.load_gather(idx_v, [lrow, kcol], mask=live)
                    hit = jnp.logical_and(vals == 0, live)
                nz = nz + jnp.sum(jnp.where(hit, jnp.float32(1.0),
                                            jnp.float32(0.0)))
            out_v[k, pl.ds(0, H)] = (acc_lo - nz * t0_lo) * scale
            out_v[k, pl.ds(H, H)] = (acc_hi - nz * t0_hi) * scale

        for d in range(D):
            dcol = jnp.full((16,), d, jnp.int32)
            for q in range(CHUNK // 16):
                krow = q * 16 + lanes
                outt_v[d, pl.ds(q * 16, 16)] = plsc.load_gather(
                    out_v, [krow, dcol])

        pltpu.sync_copy(outt_v, out_hbm.at[:, pl.ds(bc, CHUNK)])


def kernel(words, table):
    words_t = words.T
    mesh = plsc.VectorSubcoreMesh(core_axis_name="c", subcore_axis_name="s")
    cp = pltpu.CompilerParams(use_tc_tiling_on_sc=False)
    if "needs_layout_passes" in pltpu.CompilerParams.__dataclass_fields__:
        cp = dataclasses.replace(cp, needs_layout_passes=False)
    f = pl.kernel(
        _embed_mean_body,
        out_type=jax.ShapeDtypeStruct((D, B), jnp.float32),
        mesh=mesh,
        scratch_types=[
            pltpu.VMEM((L_PAD, CHUNK), jnp.int32),
            pltpu.VMEM((L * CHUNK, D), jnp.float32),
            pltpu.VMEM((CHUNK, D), jnp.float32),
            pltpu.VMEM((D, CHUNK), jnp.float32),
            pltpu.VMEM((1, D), jnp.float32),
            pltpu.SemaphoreType.DMA,
        ],
        compiler_params=cp,
    )
    return f(words_t, table).T
